# Initial kernel scaffold; baseline (speedup 1.0000x reference)
import jax
import jax.numpy as jnp
from jax.experimental import pallas as pl


def kernel(pointcloud, params):
    raise NotImplementedError("write your pallas kernel here")


# speedup vs baseline: 8.2605x; 8.2605x over previous
import functools

import jax
import jax.numpy as jnp
from jax import lax
from jax.experimental import pallas as pl
from jax.experimental.pallas import tpu as pltpu
from jax.experimental.pallas import tpu_sc as plsc

BATCH = 8
NPTS = 8192
EPSBN = 1e-5


def _fps_body(pts_ref, idx_ref, dist_ref, *, n, s, b):
    px = pts_ref[0]
    py = pts_ref[1]
    pz = pts_ref[2]
    dist_ref[...] = jnp.full((b, n), 1e10, jnp.float32)
    idx_ref[0:1, :] = jnp.zeros((1, b), jnp.int32)
    iot = lax.broadcasted_iota(jnp.int32, (b, n), 1)

    def step(i, lasts):
        eq = (iot == lasts[:, None]).astype(jnp.float32)
        cx = jnp.sum(eq * px, axis=1, keepdims=True)
        cy = jnp.sum(eq * py, axis=1, keepdims=True)
        cz = jnp.sum(eq * pz, axis=1, keepdims=True)
        dx = px - cx
        dy = py - cy
        dz = pz - cz
        d = dx * dx + dy * dy + dz * dz
        dn = jnp.minimum(dist_ref[...], d)
        dist_ref[...] = dn
        m = jnp.max(dn, axis=1, keepdims=True)
        cand = jnp.where(dn == m, iot, n)
        nxt = jnp.min(cand, axis=1).astype(jnp.int32)
        idx_ref[pl.ds(i, 1), :] = nxt[None, :]
        return nxt

    lax.fori_loop(1, s, step, jnp.zeros((b,), jnp.int32))


def _fps(ptsT3, n, s):
    return pl.pallas_call(
        functools.partial(_fps_body, n=n, s=s, b=BATCH),
        out_shape=jax.ShapeDtypeStruct((s, BATCH), jnp.int32),
        scratch_shapes=[pltpu.VMEM((BATCH, n), jnp.float32)],
    )(ptsT3)


def _make_bq(n, s, k, radius, want_idx, idx_stride):
    cw = s // 4
    kc = max(k // 16, 1)
    r2 = jnp.float32(radius * radius)
    mesh = plsc.VectorSubcoreMesh(core_axis_name="c", subcore_axis_name="s")
    outs = [jax.ShapeDtypeStruct((BATCH * 3 * s * k,), jnp.float32),
            jax.ShapeDtypeStruct((BATCH * 3 * s,), jnp.float32)]
    if want_idx:
        outs.append(jax.ShapeDtypeStruct((BATCH * s * k,), jnp.int32))
    scratch = [pltpu.VMEM((n,), jnp.float32),
               pltpu.VMEM((n,), jnp.float32),
               pltpu.VMEM((n,), jnp.float32),
               pltpu.VMEM((cw,), jnp.int32),
               pltpu.VMEM((cw,), jnp.float32),
               pltpu.VMEM((cw,), jnp.float32),
               pltpu.VMEM((cw,), jnp.float32),
               pltpu.VMEM((k + 16,), jnp.int32),
               pltpu.VMEM((cw * k,), jnp.float32),
               pltpu.VMEM((cw * k,), jnp.float32),
               pltpu.VMEM((cw * k,), jnp.float32)]
    if want_idx:
        scratch.append(pltpu.VMEM((cw * k,), jnp.int32))

    @functools.partial(pl.kernel, mesh=mesh, out_type=tuple(outs),
                       scratch_types=scratch,
                       compiler_params=pltpu.CompilerParams(
                           needs_layout_passes=False))
    def bq(ptsT_hbm, fps_hbm, *rest):
        if want_idx:
            g_hbm, nx_hbm, idx_hbm = rest[0], rest[1], rest[2]
            px, py, pz, cidx, cxr, cyr, czr, sel, ox, oy, oz, oi = rest[3:]
        else:
            g_hbm, nx_hbm = rest[0], rest[1]
            px, py, pz, cidx, cxr, cyr, czr, sel, ox, oy, oz = rest[2:]
        wid = lax.axis_index("s") * 2 + lax.axis_index("c")
        b = wid // 4
        s0 = (wid % 4) * cw
        pltpu.sync_copy(ptsT_hbm.at[pl.ds((b * 3 + 0) * n, n)], px)
        pltpu.sync_copy(ptsT_hbm.at[pl.ds((b * 3 + 1) * n, n)], py)
        pltpu.sync_copy(ptsT_hbm.at[pl.ds((b * 3 + 2) * n, n)], pz)
        pltpu.sync_copy(fps_hbm.at[pl.ds(b * s + s0, cw)], cidx)
        for t in range(cw // 16):
            civ = cidx[pl.ds(t * 16, 16)]
            cxr[pl.ds(t * 16, 16)] = plsc.load_gather(px, [civ])
            cyr[pl.ds(t * 16, 16)] = plsc.load_gather(py, [civ])
            czr[pl.ds(t * 16, 16)] = plsc.load_gather(pz, [civ])

        def center_grp(tg, _):
            cxv = cxr[pl.ds(tg * 16, 16)]
            cyv = cyr[pl.ds(tg * 16, 16)]
            czv = czr[pl.ds(tg * 16, 16)]
            for lane in range(16):
                ci = tg * 16 + lane
                cx = cxv[lane]
                cy = cyv[lane]
                cz = czv[lane]

                def chunk(j, cnt, cx=cx, cy=cy, cz=cz):
                    xv = px[pl.ds(j * 16, 16)]
                    yv = py[pl.ds(j * 16, 16)]
                    zv = pz[pl.ds(j * 16, 16)]
                    dx = xv - cx
                    dy = yv - cy
                    dz = zv - cz
                    d2 = dx * dx + dy * dy
                    d2 = d2 + dz * dz
                    msk = d2 <= r2
                    c16 = jnp.sum(msk.astype(jnp.int32))

                    @pl.when(jnp.logical_and(cnt < k, c16 > 0))
                    def _():
                        idxs = lax.iota(jnp.int32, 16) + j * 16
                        plsc.store_compressed(sel.at[pl.ds(cnt, 16)], idxs,
                                              mask=msk)

                    return cnt + c16

                cnt = lax.fori_loop(0, n // 16, chunk, jnp.int32(0))
                first = sel[pl.ds(0, 16)][0]
                for t in range(kc):
                    pos = lax.iota(jnp.int32, 16) + t * 16
                    v = sel[pl.ds(t * 16, 16)]
                    v = jnp.where(pos < cnt, v, first)
                    gx = plsc.load_gather(px, [v]) - cx
                    gy = plsc.load_gather(py, [v]) - cy
                    gz = plsc.load_gather(pz, [v]) - cz
                    ox[pl.ds(ci * k + t * 16, 16)] = gx
                    oy[pl.ds(ci * k + t * 16, 16)] = gy
                    oz[pl.ds(ci * k + t * 16, 16)] = gz
                    if want_idx:
                        oi[pl.ds(ci * k + t * 16, 16)] = v + b * idx_stride
            return 0

        lax.fori_loop(0, cw // 16, center_grp, 0)
        pltpu.sync_copy(ox, g_hbm.at[pl.ds(((b * 3 + 0) * s + s0) * k, cw * k)])
        pltpu.sync_copy(oy, g_hbm.at[pl.ds(((b * 3 + 1) * s + s0) * k, cw * k)])
        pltpu.sync_copy(oz, g_hbm.at[pl.ds(((b * 3 + 2) * s + s0) * k, cw * k)])
        pltpu.sync_copy(cxr, nx_hbm.at[pl.ds((b * 3 + 0) * s + s0, cw)])
        pltpu.sync_copy(cyr, nx_hbm.at[pl.ds((b * 3 + 1) * s + s0, cw)])
        pltpu.sync_copy(czr, nx_hbm.at[pl.ds((b * 3 + 2) * s + s0, cw)])
        if want_idx:
            pltpu.sync_copy(oi, idx_hbm.at[pl.ds(b * s * k + s0 * k, cw * k)])

    return bq


def _make_feat_gather(rows, d, nidx):
    nw = 32
    per_w = nidx // nw
    chunk = 256
    mesh = plsc.VectorSubcoreMesh(core_axis_name="c", subcore_axis_name="s")

    @functools.partial(
        pl.kernel, mesh=mesh,
        out_type=jax.ShapeDtypeStruct((nidx, d), jnp.float32),
        scratch_types=[pltpu.VMEM((chunk,), jnp.int32),
                       pltpu.VMEM((chunk, d), jnp.float32),
                       pltpu.SemaphoreType.DMA],
        compiler_params=pltpu.CompilerParams(needs_layout_passes=False))
    def gat(table_hbm, idx_hbm, out_hbm, idx_v, rows_v, sem):
        wid = lax.axis_index("s") * 2 + lax.axis_index("c")
        base = wid * per_w
        for t in range(per_w // chunk):
            pltpu.sync_copy(idx_hbm.at[pl.ds(base + t * chunk, chunk)], idx_v)
            pltpu.async_copy(table_hbm.at[idx_v], rows_v, sem).wait()
            pltpu.sync_copy(rows_v, out_hbm.at[pl.ds(base + t * chunk, chunk)])

    return gat


def _stats_body(x_ref, w_ref, sum_ref, ssq_ref):
    @pl.when(jnp.logical_and(pl.program_id(0) == 0, pl.program_id(1) == 0))
    def _():
        sum_ref[...] = jnp.zeros_like(sum_ref)
        ssq_ref[...] = jnp.zeros_like(ssq_ref)

    y = lax.dot_general(w_ref[...], x_ref[0], (((1,), (0,)), ((), ())),
                        preferred_element_type=jnp.float32)
    sum_ref[...] += jnp.sum(y, axis=1, keepdims=True)
    ssq_ref[...] += jnp.sum(y * y, axis=1, keepdims=True)


def _apply_body(x_ref, w_ref, sum_ref, ssq_ref, g_ref, b_ref, o_ref, *,
                mtot, pool_k):
    y = lax.dot_general(w_ref[...], x_ref[0], (((1,), (0,)), ((), ())),
                        preferred_element_type=jnp.float32)
    mean = sum_ref[...] / mtot
    var = ssq_ref[...] / mtot - mean * mean
    y = (y - mean) / jnp.sqrt(var + EPSBN)
    y = jnp.maximum(y * g_ref[...] + b_ref[...], 0.0)
    if pool_k:
        co, ck = y.shape
        o_ref[...] = jnp.max(y.reshape(co, ck // pool_k, pool_k), axis=-1)[None]
    else:
        o_ref[...] = y[None]


def _mlp_stage(x, layers, pool_k):
    bb, _, m = x.shape
    ck = 8192 if m % 8192 == 0 else m
    cb = m // ck
    mtot = float(bb * m)
    for li, lyr in enumerate(layers):
        w = lyr["w"]
        co, ci = w.shape
        grid = (bb, cb)
        xspec = pl.BlockSpec((1, ci, ck), lambda b, c: (b, 0, c))
        wspec = pl.BlockSpec((co, ci), lambda b, c: (0, 0))
        vspec = pl.BlockSpec((co, 1), lambda b, c: (0, 0))
        sums, ssq = pl.pallas_call(
            _stats_body,
            grid=grid,
            in_specs=[xspec, wspec],
            out_specs=[vspec, vspec],
            out_shape=[jax.ShapeDtypeStruct((co, 1), jnp.float32),
                       jax.ShapeDtypeStruct((co, 1), jnp.float32)],
        )(x, w)
        last = li == len(layers) - 1
        pk = pool_k if last else 0
        ock = ck // pk if pk else ck
        x = pl.pallas_call(
            functools.partial(_apply_body, mtot=mtot, pool_k=pk),
            grid=grid,
            in_specs=[xspec, wspec, vspec, vspec, vspec, vspec],
            out_specs=pl.BlockSpec((1, co, ock), lambda b, c: (b, 0, c)),
            out_shape=jax.ShapeDtypeStruct((bb, co, cb * ock), jnp.float32),
        )(x, w, sums, ssq, lyr["g"][:, None], lyr["b"][:, None])
    return x


def _sa3_body(x_ref, w1, g1, b1, w2, g2, b2, w3, g3, b3, o_ref):
    x = x_ref[...]
    for w_ref, gg, bb in ((w1, g1, b1), (w2, g2, b2), (w3, g3, b3)):
        y = lax.dot_general(w_ref[...], x, (((1,), (0,)), ((), ())),
                            preferred_element_type=jnp.float32)
        mean = jnp.mean(y, axis=1, keepdims=True)
        var = jnp.mean((y - mean) ** 2, axis=1, keepdims=True)
        y = (y - mean) / jnp.sqrt(var + EPSBN)
        x = jnp.maximum(y * gg[...] + bb[...], 0.0)
    co = x.shape[0]
    o_ref[...] = jnp.max(x.reshape(co, BATCH, x.shape[1] // BATCH), axis=-1)


def _sa3(x, layers):
    args = []
    for lyr in layers:
        args += [lyr["w"], lyr["g"][:, None], lyr["b"][:, None]]
    return pl.pallas_call(
        _sa3_body,
        out_shape=jax.ShapeDtypeStruct((1024, BATCH), jnp.float32),
    )(x, *args)


def kernel(pointcloud, params):
    xyz = pointcloud[..., 0:3]
    xyzT = jnp.transpose(xyz, (0, 2, 1))
    pts3 = jnp.transpose(xyz, (2, 0, 1))

    fps1 = jnp.transpose(_fps(pts3, NPTS, 512))
    bq1 = _make_bq(NPTS, 512, 32, 0.04, False, 0)
    g1, nx1T = bq1(xyzT.reshape(-1), fps1.reshape(-1))
    g1 = g1.reshape(BATCH, 3, 512, 32)
    nx1T = nx1T.reshape(BATCH, 3, 512)
    f1 = _mlp_stage(g1.reshape(BATCH, 3, 512 * 32), params["sa1"], 32)

    nx1_3 = jnp.transpose(nx1T, (1, 0, 2))
    fps2 = jnp.transpose(_fps(nx1_3, 512, 256))
    bq2 = _make_bq(512, 256, 16, 0.1, True, 512)
    g2, nx2T, idx2 = bq2(nx1T.reshape(-1), fps2.reshape(-1))
    g2 = g2.reshape(BATCH, 3, 256, 16)
    nx2T = nx2T.reshape(BATCH, 3, 256)
    idx2 = idx2.reshape(BATCH, 256, 16)
    table = jnp.transpose(f1, (0, 2, 1)).reshape(BATCH * 512, 128)
    gat = _make_feat_gather(BATCH * 512, 128, BATCH * 256 * 16)
    gf = gat(table, idx2.reshape(-1))
    gfT = jnp.transpose(gf.reshape(BATCH, 4096, 128), (0, 2, 1))
    x2 = jnp.concatenate([g2.reshape(BATCH, 3, 4096), gfT], axis=1)
    f2 = _mlp_stage(x2, params["sa2"], 16)

    x3 = jnp.concatenate([nx2T, f2], axis=1)
    x3 = jnp.transpose(x3, (1, 0, 2)).reshape(259, BATCH * 256)
    out = _sa3(x3, params["sa3"])
    return jnp.transpose(out)[..., None]
